# Initial kernel scaffold; baseline (speedup 1.0000x reference)
import jax
import jax.numpy as jnp
from jax.experimental import pallas as pl


def kernel(h, edge_index, W, b):
    raise NotImplementedError("write your pallas kernel here")


# speedup vs baseline: 3.2012x; 3.2012x over previous
import functools

import jax
import jax.numpy as jnp
from jax import lax
from jax.experimental import pallas as pl
from jax.experimental.pallas import tpu as pltpu
from jax.experimental.pallas import tpu_sc as plsc

NC = 2
NS = 16
NW = NC * NS

N_NODES = 10000
NPAD = 10240
D = 128
LANES = 16
CHUNK = 128
SLAB = 8
ROWS_PER_SUB = NPAD // NS


def _sc_aggregate(h, src_ck, dst_ck, chunks_per_w):
    mesh = plsc.VectorSubcoreMesh(
        core_axis_name="c", subcore_axis_name="s", num_cores=NC,
        num_subcores=NS)
    n_slabs = chunks_per_w // SLAB

    @functools.partial(
        pl.kernel,
        out_type=[
            jax.ShapeDtypeStruct((NC, NPAD, D), jnp.float32),
            jax.ShapeDtypeStruct((NC, NPAD, D), jnp.float32),
        ],
        mesh=mesh,
        scratch_types=[
            pltpu.VMEM((SLAB, CHUNK), jnp.int32),
            pltpu.VMEM((SLAB, CHUNK), jnp.int32),
            pltpu.VMEM((CHUNK, D), jnp.float32),
            pltpu.VMEM((CHUNK, D), jnp.float32),
            pltpu.VMEM_SHARED((NPAD, D), jnp.float32),
            pltpu.SemaphoreType.DMA,
        ],
    )
    def sc_kernel(h_hbm, src_hbm, dst_hbm, agg_out, deg_out,
                  src_v, dst_v, rows_v, ones_v, acc_sh, sem):
        c = lax.axis_index("c")
        s = lax.axis_index("s")
        w = c * NS + s
        row0 = pl.multiple_of(s * ROWS_PER_SUB, CHUNK)
        zero16 = jnp.zeros((LANES,), jnp.float32)
        one16 = jnp.ones((LANES,), jnp.float32)

        @pl.loop(0, CHUNK)
        def _(i):
            for k in range(D // LANES):
                rows_v[i, pl.ds(k * LANES, LANES)] = zero16
                ones_v[i, pl.ds(k * LANES, LANES)] = one16

        @pl.loop(0, ROWS_PER_SUB // CHUNK)
        def _(t):
            r = pl.multiple_of(row0 + t * CHUNK, CHUNK)
            pltpu.async_copy(rows_v, acc_sh.at[pl.ds(r, CHUNK)], sem).wait()

        plsc.subcore_barrier()

        @pl.loop(0, n_slabs)
        def _(ss):
            base = pl.multiple_of(w * chunks_per_w + ss * SLAB, SLAB)
            pltpu.sync_copy(src_hbm.at[pl.ds(base, SLAB)], src_v)
            pltpu.sync_copy(dst_hbm.at[pl.ds(base, SLAB)], dst_v)

            @pl.loop(0, SLAB)
            def _(j):
                pltpu.async_copy(h_hbm.at[src_v.at[j]], rows_v, sem).wait()
                pltpu.async_copy(rows_v, acc_sh.at[dst_v.at[j]], sem,
                                 add=True).wait()

        plsc.subcore_barrier()
        pltpu.async_copy(acc_sh.at[pl.ds(row0, ROWS_PER_SUB)],
                         agg_out.at[c, pl.ds(row0, ROWS_PER_SUB)],
                         sem).wait()

        @pl.loop(0, CHUNK)
        def _(i):
            for k in range(D // LANES):
                rows_v[i, pl.ds(k * LANES, LANES)] = zero16

        @pl.loop(0, ROWS_PER_SUB // CHUNK)
        def _(t):
            r = pl.multiple_of(row0 + t * CHUNK, CHUNK)
            pltpu.async_copy(rows_v, acc_sh.at[pl.ds(r, CHUNK)], sem).wait()

        plsc.subcore_barrier()

        @pl.loop(0, n_slabs)
        def _(ss):
            base = pl.multiple_of(w * chunks_per_w + ss * SLAB, SLAB)
            pltpu.sync_copy(dst_hbm.at[pl.ds(base, SLAB)], dst_v)

            @pl.loop(0, SLAB)
            def _(j):
                pltpu.async_copy(ones_v, acc_sh.at[dst_v.at[j]], sem,
                                 add=True).wait()

        plsc.subcore_barrier()
        pltpu.async_copy(acc_sh.at[pl.ds(row0, ROWS_PER_SUB)],
                         deg_out.at[c, pl.ds(row0, ROWS_PER_SUB)],
                         sem).wait()

    return sc_kernel(h, src_ck, dst_ck)


def _tc_finish_body(agg_ref, deg_ref, wt_ref, b_ref, o_ref):
    agg = agg_ref[0] + agg_ref[1]
    deg = deg_ref[0, :, 0:1] + deg_ref[1, :, 0:1]
    h_neigh = agg / jnp.maximum(deg, 1.0)
    o_ref[...] = (
        jnp.dot(h_neigh, wt_ref[...], preferred_element_type=jnp.float32)
        + b_ref[...])


def _tc_finish(agg, deg, w_t, b2):
    br = 2000
    grid = (N_NODES // br,)
    return pl.pallas_call(
        _tc_finish_body,
        grid=grid,
        in_specs=[
            pl.BlockSpec((NC, br, D), lambda i: (0, i, 0)),
            pl.BlockSpec((NC, br, D), lambda i: (0, i, 0)),
            pl.BlockSpec((D, D), lambda i: (0, 0)),
            pl.BlockSpec((1, D), lambda i: (0, 0)),
        ],
        out_specs=pl.BlockSpec((br, D), lambda i: (i, 0)),
        out_shape=jax.ShapeDtypeStruct((N_NODES, D), jnp.float32),
    )(agg, deg, w_t, b2)


@jax.jit
def kernel(h, edge_index, W, b):
    n_edges = edge_index.shape[1]
    e_pad = -(-n_edges // (NW * CHUNK * SLAB)) * (NW * CHUNK * SLAB)
    n_fake = e_pad - n_edges
    chunks_per_w = e_pad // (NW * CHUNK)

    src = edge_index[0].astype(jnp.int32)
    dst = edge_index[1].astype(jnp.int32)
    if n_fake:
        src = jnp.concatenate([src, jnp.zeros((n_fake,), jnp.int32)])
        fake_dst = N_NODES + (jnp.arange(n_fake, dtype=jnp.int32)
                              % (NPAD - N_NODES))
        dst = jnp.concatenate([dst, fake_dst])
    src_ck = src.reshape(NW * chunks_per_w, CHUNK)
    dst_ck = dst.reshape(NW * chunks_per_w, CHUNK)

    agg, deg = _sc_aggregate(h, src_ck, dst_ck, chunks_per_w)
    return _tc_finish(agg, deg, W.T, b.reshape(1, D))


# speedup vs baseline: 3.4157x; 1.0670x over previous
import functools

import jax
import jax.numpy as jnp
from jax import lax
from jax.experimental import pallas as pl
from jax.experimental.pallas import tpu as pltpu
from jax.experimental.pallas import tpu_sc as plsc

NC = 2
NS = 16
NW = NC * NS

N_NODES = 10000
NPAD = 10240
D = 128
LANES = 16
CHUNK = 128
SLAB = 8
ROWS_PER_SUB = NPAD // NS


def _sc_aggregate(h, src_ck, dst_ck, chunks_per_w):
    mesh = plsc.VectorSubcoreMesh(
        core_axis_name="c", subcore_axis_name="s", num_cores=NC,
        num_subcores=NS)
    n_slabs = chunks_per_w // SLAB

    @functools.partial(
        pl.kernel,
        out_type=[
            jax.ShapeDtypeStruct((NC, NPAD, D), jnp.float32),
            jax.ShapeDtypeStruct((NC, NPAD, D), jnp.float32),
        ],
        mesh=mesh,
        scratch_types=[
            pltpu.VMEM((SLAB, CHUNK), jnp.int32),
            pltpu.VMEM((SLAB, CHUNK), jnp.int32),
            pltpu.VMEM((CHUNK, D), jnp.float32),
            pltpu.VMEM((CHUNK, D), jnp.float32),
            pltpu.VMEM_SHARED((NPAD, D), jnp.float32),
            pltpu.SemaphoreType.DMA,
            pltpu.SemaphoreType.DMA,
        ],
    )
    def sc_kernel(h_hbm, src_hbm, dst_hbm, agg_out, deg_out,
                  src_v, dst_v, rows_v, rows_b, acc_sh, sem, sem_s):
        c = lax.axis_index("c")
        s = lax.axis_index("s")
        w = c * NS + s
        row0 = pl.multiple_of(s * ROWS_PER_SUB, CHUNK)
        zero16 = jnp.zeros((LANES,), jnp.float32)
        one16 = jnp.ones((LANES,), jnp.float32)

        @pl.loop(0, CHUNK)
        def _(i):
            for k in range(D // LANES):
                rows_v[i, pl.ds(k * LANES, LANES)] = zero16

        @pl.loop(0, ROWS_PER_SUB // CHUNK)
        def _(t):
            r = pl.multiple_of(row0 + t * CHUNK, CHUNK)
            pltpu.async_copy(rows_v, acc_sh.at[pl.ds(r, CHUNK)], sem).wait()

        plsc.subcore_barrier()

        @pl.loop(0, n_slabs)
        def _(ss):
            base = pl.multiple_of(w * chunks_per_w + ss * SLAB, SLAB)
            pltpu.sync_copy(src_hbm.at[pl.ds(base, SLAB)], src_v)
            pltpu.sync_copy(dst_hbm.at[pl.ds(base, SLAB)], dst_v)

            bufs = (rows_v, rows_b)
            g = pltpu.async_copy(h_hbm.at[src_v.at[0]], bufs[0], sem)
            for j in range(SLAB):
                g.wait()
                if j + 1 < SLAB:
                    g = pltpu.async_copy(h_hbm.at[src_v.at[j + 1]],
                                         bufs[(j + 1) % 2], sem)
                pltpu.async_copy(bufs[j % 2], acc_sh.at[dst_v.at[j]],
                                 sem_s, add=True).wait()

        plsc.subcore_barrier()
        pltpu.async_copy(acc_sh.at[pl.ds(row0, ROWS_PER_SUB)],
                         agg_out.at[c, pl.ds(row0, ROWS_PER_SUB)],
                         sem).wait()

        @pl.loop(0, CHUNK)
        def _(i):
            for k in range(D // LANES):
                rows_v[i, pl.ds(k * LANES, LANES)] = zero16

        @pl.loop(0, ROWS_PER_SUB // CHUNK)
        def _(t):
            r = pl.multiple_of(row0 + t * CHUNK, CHUNK)
            pltpu.async_copy(rows_v, acc_sh.at[pl.ds(r, CHUNK)], sem).wait()

        @pl.loop(0, CHUNK)
        def _(i):
            for k in range(D // LANES):
                rows_v[i, pl.ds(k * LANES, LANES)] = one16

        plsc.subcore_barrier()

        @pl.loop(0, n_slabs)
        def _(ss):
            base = pl.multiple_of(w * chunks_per_w + ss * SLAB, SLAB)
            pltpu.sync_copy(dst_hbm.at[pl.ds(base, SLAB)], dst_v)

            descs = [
                pltpu.async_copy(rows_v, acc_sh.at[dst_v.at[j]], sem_s,
                                 add=True)
                for j in range(SLAB)
            ]
            for d in descs:
                d.wait()

        plsc.subcore_barrier()
        pltpu.async_copy(acc_sh.at[pl.ds(row0, ROWS_PER_SUB)],
                         deg_out.at[c, pl.ds(row0, ROWS_PER_SUB)],
                         sem).wait()

    return sc_kernel(h, src_ck, dst_ck)


def _tc_finish_body(agg_ref, deg_ref, wt_ref, b_ref, o_ref):
    agg = agg_ref[0] + agg_ref[1]
    deg = deg_ref[0, :, 0:1] + deg_ref[1, :, 0:1]
    h_neigh = agg / jnp.maximum(deg, 1.0)
    o_ref[...] = (
        jnp.dot(h_neigh, wt_ref[...], preferred_element_type=jnp.float32)
        + b_ref[...])


def _tc_finish(agg, deg, w_t, b2):
    br = 2000
    grid = (N_NODES // br,)
    return pl.pallas_call(
        _tc_finish_body,
        grid=grid,
        in_specs=[
            pl.BlockSpec((NC, br, D), lambda i: (0, i, 0)),
            pl.BlockSpec((NC, br, D), lambda i: (0, i, 0)),
            pl.BlockSpec((D, D), lambda i: (0, 0)),
            pl.BlockSpec((1, D), lambda i: (0, 0)),
        ],
        out_specs=pl.BlockSpec((br, D), lambda i: (i, 0)),
        out_shape=jax.ShapeDtypeStruct((N_NODES, D), jnp.float32),
    )(agg, deg, w_t, b2)


@jax.jit
def kernel(h, edge_index, W, b):
    n_edges = edge_index.shape[1]
    e_pad = -(-n_edges // (NW * CHUNK * SLAB)) * (NW * CHUNK * SLAB)
    n_fake = e_pad - n_edges
    chunks_per_w = e_pad // (NW * CHUNK)

    src = edge_index[0].astype(jnp.int32)
    dst = edge_index[1].astype(jnp.int32)
    if n_fake:
        src = jnp.concatenate([src, jnp.zeros((n_fake,), jnp.int32)])
        fake_dst = N_NODES + (jnp.arange(n_fake, dtype=jnp.int32)
                              % (NPAD - N_NODES))
        dst = jnp.concatenate([dst, fake_dst])
    src_ck = src.reshape(NW * chunks_per_w, CHUNK)
    dst_ck = dst.reshape(NW * chunks_per_w, CHUNK)

    agg, deg = _sc_aggregate(h, src_ck, dst_ck, chunks_per_w)
    return _tc_finish(agg, deg, W.T, b.reshape(1, D))


# speedup vs baseline: 3.4511x; 1.0104x over previous
import functools

import jax
import jax.numpy as jnp
from jax import lax
from jax.experimental import pallas as pl
from jax.experimental.pallas import tpu as pltpu
from jax.experimental.pallas import tpu_sc as plsc

NC = 2
NS = 16
NW = NC * NS

N_NODES = 10000
NPAD = 10240
D = 128
LANES = 16
CHUNK = 128
SLAB = 16
ROWS_PER_SUB = NPAD // NS


def _sc_aggregate(h, src_ck, dst_ck, chunks_per_w):
    mesh = plsc.VectorSubcoreMesh(
        core_axis_name="c", subcore_axis_name="s", num_cores=NC,
        num_subcores=NS)
    n_slabs = chunks_per_w // SLAB

    @functools.partial(
        pl.kernel,
        out_type=[
            jax.ShapeDtypeStruct((NC, NPAD, D), jnp.float32),
            jax.ShapeDtypeStruct((NC, NPAD, D), jnp.float32),
        ],
        mesh=mesh,
        scratch_types=[
            pltpu.VMEM((SLAB, CHUNK), jnp.int32),
            pltpu.VMEM((SLAB, CHUNK), jnp.int32),
            pltpu.VMEM((CHUNK, D), jnp.float32),
            pltpu.VMEM((CHUNK, D), jnp.float32),
            pltpu.VMEM_SHARED((NPAD, D), jnp.float32),
            pltpu.SemaphoreType.DMA,
            pltpu.SemaphoreType.DMA,
            pltpu.SemaphoreType.DMA,
        ],
    )
    def sc_kernel(h_hbm, src_hbm, dst_hbm, agg_out, deg_out,
                  src_v, dst_v, rows_v, rows_b, acc_sh, sem, sem_s, sem_s2):
        c = lax.axis_index("c")
        s = lax.axis_index("s")
        w = c * NS + s
        row0 = pl.multiple_of(s * ROWS_PER_SUB, CHUNK)
        zero16 = jnp.zeros((LANES,), jnp.float32)
        one16 = jnp.ones((LANES,), jnp.float32)

        @pl.loop(0, CHUNK)
        def _(i):
            for k in range(D // LANES):
                rows_v[i, pl.ds(k * LANES, LANES)] = zero16

        @pl.loop(0, ROWS_PER_SUB // CHUNK)
        def _(t):
            r = pl.multiple_of(row0 + t * CHUNK, CHUNK)
            pltpu.async_copy(rows_v, acc_sh.at[pl.ds(r, CHUNK)], sem).wait()

        plsc.subcore_barrier()

        @pl.loop(0, n_slabs)
        def _(ss):
            base = pl.multiple_of(w * chunks_per_w + ss * SLAB, SLAB)
            pltpu.sync_copy(src_hbm.at[pl.ds(base, SLAB)], src_v)
            pltpu.sync_copy(dst_hbm.at[pl.ds(base, SLAB)], dst_v)

            bufs = (rows_v, rows_b)
            ssems = (sem_s, sem_s2)
            g = pltpu.async_copy(h_hbm.at[src_v.at[0]], bufs[0], sem)
            prev_s = None
            for j in range(SLAB):
                g.wait()
                s = pltpu.async_copy(bufs[j % 2], acc_sh.at[dst_v.at[j]],
                                     ssems[j % 2], add=True)
                if prev_s is not None:
                    prev_s.wait()
                if j + 1 < SLAB:
                    g = pltpu.async_copy(h_hbm.at[src_v.at[j + 1]],
                                         bufs[(j + 1) % 2], sem)
                prev_s = s
            prev_s.wait()

        plsc.subcore_barrier()
        pltpu.async_copy(acc_sh.at[pl.ds(row0, ROWS_PER_SUB)],
                         agg_out.at[c, pl.ds(row0, ROWS_PER_SUB)],
                         sem).wait()

        @pl.loop(0, CHUNK)
        def _(i):
            for k in range(D // LANES):
                rows_v[i, pl.ds(k * LANES, LANES)] = zero16

        @pl.loop(0, ROWS_PER_SUB // CHUNK)
        def _(t):
            r = pl.multiple_of(row0 + t * CHUNK, CHUNK)
            pltpu.async_copy(rows_v, acc_sh.at[pl.ds(r, CHUNK)], sem).wait()

        @pl.loop(0, CHUNK)
        def _(i):
            for k in range(D // LANES):
                rows_v[i, pl.ds(k * LANES, LANES)] = one16

        plsc.subcore_barrier()

        @pl.loop(0, n_slabs)
        def _(ss):
            base = pl.multiple_of(w * chunks_per_w + ss * SLAB, SLAB)
            pltpu.sync_copy(dst_hbm.at[pl.ds(base, SLAB)], dst_v)

            descs = [
                pltpu.async_copy(rows_v, acc_sh.at[dst_v.at[j]], sem_s,
                                 add=True)
                for j in range(SLAB)
            ]
            for d in descs:
                d.wait()

        plsc.subcore_barrier()
        pltpu.async_copy(acc_sh.at[pl.ds(row0, ROWS_PER_SUB)],
                         deg_out.at[c, pl.ds(row0, ROWS_PER_SUB)],
                         sem).wait()

    return sc_kernel(h, src_ck, dst_ck)


def _tc_finish_body(agg_ref, deg_ref, wt_ref, b_ref, o_ref):
    agg = agg_ref[0] + agg_ref[1]
    deg = deg_ref[0, :, 0:1] + deg_ref[1, :, 0:1]
    h_neigh = agg / jnp.maximum(deg, 1.0)
    o_ref[...] = (
        jnp.dot(h_neigh, wt_ref[...], preferred_element_type=jnp.float32)
        + b_ref[...])


def _tc_finish(agg, deg, w_t, b2):
    br = 2000
    grid = (N_NODES // br,)
    return pl.pallas_call(
        _tc_finish_body,
        grid=grid,
        in_specs=[
            pl.BlockSpec((NC, br, D), lambda i: (0, i, 0)),
            pl.BlockSpec((NC, br, D), lambda i: (0, i, 0)),
            pl.BlockSpec((D, D), lambda i: (0, 0)),
            pl.BlockSpec((1, D), lambda i: (0, 0)),
        ],
        out_specs=pl.BlockSpec((br, D), lambda i: (i, 0)),
        out_shape=jax.ShapeDtypeStruct((N_NODES, D), jnp.float32),
    )(agg, deg, w_t, b2)


@jax.jit
def kernel(h, edge_index, W, b):
    n_edges = edge_index.shape[1]
    e_pad = -(-n_edges // (NW * CHUNK * SLAB)) * (NW * CHUNK * SLAB)
    n_fake = e_pad - n_edges
    chunks_per_w = e_pad // (NW * CHUNK)

    src = edge_index[0].astype(jnp.int32)
    dst = edge_index[1].astype(jnp.int32)
    if n_fake:
        src = jnp.concatenate([src, jnp.zeros((n_fake,), jnp.int32)])
        fake_dst = N_NODES + (jnp.arange(n_fake, dtype=jnp.int32)
                              % (NPAD - N_NODES))
        dst = jnp.concatenate([dst, fake_dst])
    src_ck = src.reshape(NW * chunks_per_w, CHUNK)
    dst_ck = dst.reshape(NW * chunks_per_w, CHUNK)

    agg, deg = _sc_aggregate(h, src_ck, dst_ck, chunks_per_w)
    return _tc_finish(agg, deg, W.T, b.reshape(1, D))
